# Initial kernel scaffold; baseline (speedup 1.0000x reference)
import jax
import jax.numpy as jnp
from jax.experimental import pallas as pl


def kernel(x, cls_label, w1, b1, wg, bg, wb, bb):
    raise NotImplementedError("write your pallas kernel here")


# speedup vs baseline: 1.1449x; 1.1449x over previous
import functools

import jax
import jax.numpy as jnp
from jax import lax
from jax.experimental import pallas as pl
from jax.experimental.pallas import tpu as pltpu


def _stats_kernel(x_ref, sum_ref, sq_ref):
    x = x_ref[...]
    s = jnp.sum(x, axis=1, keepdims=True)
    q = jnp.sum(x * x, axis=1, keepdims=True)
    c = s.shape[0]
    sum_ref[...] = s.T.reshape(1, 1, c)
    sq_ref[...] = q.T.reshape(1, 1, c)


def _apply_kernel(sum_ref, sq_ref, h_ref, wg_ref, bg_ref, wb_ref, bb_ref,
                  x_ref, o_ref, *, inv_cnt):
    i = pl.program_id(0)
    s = jnp.sum(sum_ref[...], axis=(0, 1))
    q = jnp.sum(sq_ref[...], axis=(0, 1))
    mean = (s * inv_cnt)[None, :]
    var = (q * inv_cnt)[None, :] - mean * mean
    rstd = lax.rsqrt(var + 1e-5)
    h = h_ref[pl.ds(i, 1), :]
    gamma = jnp.dot(h, wg_ref[...],
                    preferred_element_type=jnp.float32) + bg_ref[...]
    beta = jnp.dot(h, wb_ref[...],
                   preferred_element_type=jnp.float32) + bb_ref[...]
    scale = rstd * (1.0 + gamma)
    bias = beta - mean * scale
    o_ref[...] = x_ref[...] * scale.T + bias.T


@jax.jit
def _cond_batch_norm(x, cls_label, w1, b1, wg, bg, wb, bb):
    n, c, hgt, wid = x.shape
    hw = hgt * wid
    rows = n * c
    x2 = x.astype(jnp.float32).reshape(rows, hw)

    hact = jax.nn.relu(w1[cls_label] + b1)

    stats_spec = pl.BlockSpec((1, 1, c), lambda i: (i, 0, 0))
    sum3, sq3 = pl.pallas_call(
        _stats_kernel,
        out_shape=(jax.ShapeDtypeStruct((n, 1, c), jnp.float32),
                   jax.ShapeDtypeStruct((n, 1, c), jnp.float32)),
        grid=(n,),
        in_specs=[pl.BlockSpec((c, hw), lambda i: (i, 0))],
        out_specs=(stats_spec, stats_spec),
        compiler_params=pltpu.CompilerParams(
            dimension_semantics=("parallel",),
            vmem_limit_bytes=int(40 << 20)),
        cost_estimate=pl.CostEstimate(
            flops=3 * rows * hw, transcendentals=0,
            bytes_accessed=rows * hw * 4 + 8 * rows),
    )(x2)

    const3 = pl.BlockSpec((n, 1, c), lambda i: (0, 0, 0))
    tile = pl.BlockSpec((c, hw), lambda i: (i, 0))
    out2 = pl.pallas_call(
        functools.partial(_apply_kernel, inv_cnt=1.0 / float(n * hw)),
        out_shape=jax.ShapeDtypeStruct((rows, hw), jnp.float32),
        grid=(n,),
        in_specs=[const3, const3,
                  pl.BlockSpec(hact.shape, lambda i: (0, 0)),
                  pl.BlockSpec(wg.shape, lambda i: (0, 0)),
                  pl.BlockSpec(bg.shape, lambda i: (0, 0)),
                  pl.BlockSpec(wb.shape, lambda i: (0, 0)),
                  pl.BlockSpec(bb.shape, lambda i: (0, 0)),
                  tile],
        out_specs=tile,
        compiler_params=pltpu.CompilerParams(
            dimension_semantics=("parallel",),
            vmem_limit_bytes=int(48 << 20)),
        cost_estimate=pl.CostEstimate(
            flops=2 * rows * hw, transcendentals=c,
            bytes_accessed=2 * rows * hw * 4 + 8 * rows),
    )(sum3, sq3, hact, wg, bg, wb, bb, x2)
    return out2.reshape(n, c, hgt, wid)


def kernel(x, cls_label, w1, b1, wg, bg, wb, bb):
    return _cond_batch_norm(x, cls_label, w1, b1, wg, bg, wb, bb)


# speedup vs baseline: 1.8520x; 1.6176x over previous
import functools

import jax
import jax.numpy as jnp
from jax import lax
from jax.experimental import pallas as pl
from jax.experimental.pallas import tpu as pltpu


def _stats_kernel(x_ref, sum_ref, sq_ref):
    x = x_ref[...]
    s = jnp.sum(x, axis=(1, 2))
    q = jnp.sum(x * x, axis=(1, 2))
    c = s.shape[0]
    sum_ref[...] = s.reshape(1, 1, c)
    sq_ref[...] = q.reshape(1, 1, c)


def _apply_kernel(sum_ref, sq_ref, h_ref, wg_ref, bg_ref, wb_ref, bb_ref,
                  x_ref, o_ref, *, inv_cnt):
    i = pl.program_id(0)
    s = jnp.sum(sum_ref[...], axis=(0, 1))
    q = jnp.sum(sq_ref[...], axis=(0, 1))
    mean = (s * inv_cnt)[None, :]
    var = (q * inv_cnt)[None, :] - mean * mean
    rstd = lax.rsqrt(var + 1e-5)
    h = h_ref[pl.ds(i, 1), :]
    gamma = jnp.dot(h, wg_ref[...],
                    preferred_element_type=jnp.float32) + bg_ref[...]
    beta = jnp.dot(h, wb_ref[...],
                   preferred_element_type=jnp.float32) + bb_ref[...]
    scale = rstd * (1.0 + gamma)
    bias = beta - mean * scale
    c = scale.shape[1]
    o_ref[...] = (x_ref[...] * scale.reshape(c, 1, 1)
                  + bias.reshape(c, 1, 1))


@jax.jit
def _cond_batch_norm(x, cls_label, w1, b1, wg, bg, wb, bb):
    n, c, hgt, wid = x.shape
    rows = n * c
    x3 = x.astype(jnp.float32).reshape(rows, hgt, wid)

    hact = jax.nn.relu(w1[cls_label] + b1)

    stats_spec = pl.BlockSpec((1, 1, c), lambda i: (i, 0, 0))
    sum3, sq3 = pl.pallas_call(
        _stats_kernel,
        out_shape=(jax.ShapeDtypeStruct((n, 1, c), jnp.float32),
                   jax.ShapeDtypeStruct((n, 1, c), jnp.float32)),
        grid=(n,),
        in_specs=[pl.BlockSpec((c, hgt, wid), lambda i: (i, 0, 0))],
        out_specs=(stats_spec, stats_spec),
        compiler_params=pltpu.CompilerParams(
            dimension_semantics=("parallel",),
            vmem_limit_bytes=int(48 << 20)),
        cost_estimate=pl.CostEstimate(
            flops=3 * rows * hgt * wid, transcendentals=0,
            bytes_accessed=rows * hgt * wid * 4 + 8 * rows),
    )(x3)

    const3 = pl.BlockSpec((n, 1, c), lambda i: (0, 0, 0))
    tile = pl.BlockSpec((c, hgt, wid), lambda i: (i, 0, 0))
    out3 = pl.pallas_call(
        functools.partial(_apply_kernel, inv_cnt=1.0 / float(n * hgt * wid)),
        out_shape=jax.ShapeDtypeStruct((rows, hgt, wid), jnp.float32),
        grid=(n,),
        in_specs=[const3, const3,
                  pl.BlockSpec(hact.shape, lambda i: (0, 0)),
                  pl.BlockSpec(wg.shape, lambda i: (0, 0)),
                  pl.BlockSpec(bg.shape, lambda i: (0, 0)),
                  pl.BlockSpec(wb.shape, lambda i: (0, 0)),
                  pl.BlockSpec(bb.shape, lambda i: (0, 0)),
                  tile],
        out_specs=tile,
        compiler_params=pltpu.CompilerParams(
            dimension_semantics=("parallel",),
            vmem_limit_bytes=int(48 << 20)),
        cost_estimate=pl.CostEstimate(
            flops=2 * rows * hgt * wid, transcendentals=c,
            bytes_accessed=2 * rows * hgt * wid * 4 + 8 * rows),
    )(sum3, sq3, hact, wg, bg, wb, bb, x3)
    return out3.reshape(n, c, hgt, wid)


def kernel(x, cls_label, w1, b1, wg, bg, wb, bb):
    return _cond_batch_norm(x, cls_label, w1, b1, wg, bg, wb, bb)


# speedup vs baseline: 6.6330x; 3.5815x over previous
import functools

import jax
import jax.numpy as jnp
from jax import lax
from jax.experimental import pallas as pl
from jax.experimental.pallas import tpu as pltpu


def _stats_kernel(x_ref, sum_ref, sq_ref):
    x = x_ref[...]
    s = jnp.sum(x, axis=0, keepdims=True)
    q = jnp.sum(x * x, axis=0, keepdims=True)
    sum_ref[...] = s[None]
    sq_ref[...] = q[None]


def _apply_kernel(sum_ref, sq_ref, h_ref, wg_ref, bg_ref, wb_ref, bb_ref,
                  x_ref, o_ref, *, inv_cnt):
    i = pl.program_id(0)
    s = jnp.sum(sum_ref[...], axis=(0, 1))
    q = jnp.sum(sq_ref[...], axis=(0, 1))
    mean = (s * inv_cnt)[None, :]
    var = (q * inv_cnt)[None, :] - mean * mean
    rstd = lax.rsqrt(var + 1e-5)
    h = h_ref[pl.ds(i, 1), :]
    gamma = jnp.dot(h, wg_ref[...],
                    preferred_element_type=jnp.float32) + bg_ref[...]
    beta = jnp.dot(h, wb_ref[...],
                   preferred_element_type=jnp.float32) + bb_ref[...]
    scale = rstd * (1.0 + gamma)
    bias = beta - mean * scale
    o_ref[...] = x_ref[...] * scale + bias


@jax.jit
def _cond_batch_norm(x, cls_label, w1, b1, wg, bg, wb, bb):
    n, c, hgt, wid = x.shape
    hw = hgt * wid
    xr = x.astype(jnp.float32).transpose(0, 2, 3, 1).reshape(n * hw, c)

    hact = jax.nn.relu(w1[cls_label] + b1)

    stats_spec = pl.BlockSpec((1, 1, c), lambda i: (i, 0, 0))
    sum3, sq3 = pl.pallas_call(
        _stats_kernel,
        out_shape=(jax.ShapeDtypeStruct((n, 1, c), jnp.float32),
                   jax.ShapeDtypeStruct((n, 1, c), jnp.float32)),
        grid=(n,),
        in_specs=[pl.BlockSpec((hw, c), lambda i: (i, 0))],
        out_specs=(stats_spec, stats_spec),
        compiler_params=pltpu.CompilerParams(
            dimension_semantics=("parallel",),
            vmem_limit_bytes=int(48 << 20)),
        cost_estimate=pl.CostEstimate(
            flops=3 * n * hw * c, transcendentals=0,
            bytes_accessed=n * hw * c * 4 + 8 * n * c),
    )(xr)

    const3 = pl.BlockSpec((n, 1, c), lambda i: (0, 0, 0))
    tile = pl.BlockSpec((hw, c), lambda i: (i, 0))
    out2 = pl.pallas_call(
        functools.partial(_apply_kernel, inv_cnt=1.0 / float(n * hw)),
        out_shape=jax.ShapeDtypeStruct((n * hw, c), jnp.float32),
        grid=(n,),
        in_specs=[const3, const3,
                  pl.BlockSpec(hact.shape, lambda i: (0, 0)),
                  pl.BlockSpec(wg.shape, lambda i: (0, 0)),
                  pl.BlockSpec(bg.shape, lambda i: (0, 0)),
                  pl.BlockSpec(wb.shape, lambda i: (0, 0)),
                  pl.BlockSpec(bb.shape, lambda i: (0, 0)),
                  tile],
        out_specs=tile,
        compiler_params=pltpu.CompilerParams(
            dimension_semantics=("parallel",),
            vmem_limit_bytes=int(48 << 20)),
        cost_estimate=pl.CostEstimate(
            flops=2 * n * hw * c, transcendentals=c,
            bytes_accessed=2 * n * hw * c * 4 + 8 * n * c),
    )(sum3, sq3, hact, wg, bg, wb, bb, xr)
    return out2.reshape(n, hgt, wid, c).transpose(0, 3, 1, 2)


def kernel(x, cls_label, w1, b1, wg, bg, wb, bb):
    return _cond_batch_norm(x, cls_label, w1, b1, wg, bg, wb, bb)


# speedup vs baseline: 6.6742x; 1.0062x over previous
import functools

import jax
import jax.numpy as jnp
from jax import lax
from jax.experimental import pallas as pl
from jax.experimental.pallas import tpu as pltpu


def _stats_kernel(x_ref, sum_ref, sq_ref):
    x = x_ref[...]
    s = jnp.sum(x, axis=0, keepdims=True)
    q = jnp.sum(x * x, axis=0, keepdims=True)
    sum_ref[...] = s[None]
    sq_ref[...] = q[None]


def _apply_kernel(cls_ref, sum_ref, sq_ref, w1_ref, b1_ref, wg_ref, bg_ref,
                  wb_ref, bb_ref, x_ref, o_ref, *, inv_cnt):
    i = pl.program_id(0)
    s = jnp.sum(sum_ref[...], axis=(0, 1))
    q = jnp.sum(sq_ref[...], axis=(0, 1))
    mean = (s * inv_cnt)[None, :]
    var = (q * inv_cnt)[None, :] - mean * mean
    rstd = lax.rsqrt(var + 1e-5)
    lab = cls_ref[i]
    h = jnp.maximum(w1_ref[pl.ds(lab, 1), :] + b1_ref[...], 0.0)
    gamma = jnp.dot(h, wg_ref[...],
                    preferred_element_type=jnp.float32) + bg_ref[...]
    beta = jnp.dot(h, wb_ref[...],
                   preferred_element_type=jnp.float32) + bb_ref[...]
    scale = rstd * (1.0 + gamma)
    bias = beta - mean * scale
    o_ref[...] = x_ref[...] * scale + bias


@jax.jit
def _cond_batch_norm(x, cls_label, w1, b1, wg, bg, wb, bb):
    n, c, hgt, wid = x.shape
    hw = hgt * wid
    xr = x.astype(jnp.float32).transpose(0, 2, 3, 1).reshape(n * hw, c)

    stats_spec = pl.BlockSpec((1, 1, c), lambda i: (i, 0, 0))
    sum3, sq3 = pl.pallas_call(
        _stats_kernel,
        out_shape=(jax.ShapeDtypeStruct((n, 1, c), jnp.float32),
                   jax.ShapeDtypeStruct((n, 1, c), jnp.float32)),
        grid=(n,),
        in_specs=[pl.BlockSpec((hw, c), lambda i: (i, 0))],
        out_specs=(stats_spec, stats_spec),
        compiler_params=pltpu.CompilerParams(
            dimension_semantics=("parallel",),
            vmem_limit_bytes=int(48 << 20)),
        cost_estimate=pl.CostEstimate(
            flops=3 * n * hw * c, transcendentals=0,
            bytes_accessed=n * hw * c * 4 + 8 * n * c),
    )(xr)

    const3 = pl.BlockSpec((n, 1, c), lambda i: (0, 0, 0))
    tile = pl.BlockSpec((hw, c), lambda i: (i, 0))
    constp = pl.BlockSpec((n, 1, c), lambda i, *_: (0, 0, 0))
    tilep = pl.BlockSpec((hw, c), lambda i, *_: (i, 0))
    grid_spec = pltpu.PrefetchScalarGridSpec(
        num_scalar_prefetch=1,
        grid=(n,),
        in_specs=[constp, constp,
                  pl.BlockSpec(w1.shape, lambda i, *_: (0, 0)),
                  pl.BlockSpec(b1.shape, lambda i, *_: (0, 0)),
                  pl.BlockSpec(wg.shape, lambda i, *_: (0, 0)),
                  pl.BlockSpec(bg.shape, lambda i, *_: (0, 0)),
                  pl.BlockSpec(wb.shape, lambda i, *_: (0, 0)),
                  pl.BlockSpec(bb.shape, lambda i, *_: (0, 0)),
                  tilep],
        out_specs=tilep,
    )
    out2 = pl.pallas_call(
        functools.partial(_apply_kernel, inv_cnt=1.0 / float(n * hw)),
        out_shape=jax.ShapeDtypeStruct((n * hw, c), jnp.float32),
        grid_spec=grid_spec,
        compiler_params=pltpu.CompilerParams(
            dimension_semantics=("parallel",),
            vmem_limit_bytes=int(48 << 20)),
        cost_estimate=pl.CostEstimate(
            flops=2 * n * hw * c, transcendentals=c,
            bytes_accessed=2 * n * hw * c * 4 + 8 * n * c),
    )(cls_label, sum3, sq3, w1, b1, wg, bg, wb, bb, xr)
    return out2.reshape(n, hgt, wid, c).transpose(0, 3, 1, 2)


def kernel(x, cls_label, w1, b1, wg, bg, wb, bb):
    return _cond_batch_norm(x, cls_label, w1, b1, wg, bg, wb, bb)
